# Initial kernel scaffold; baseline (speedup 1.0000x reference)
import jax
import jax.numpy as jnp
from jax.experimental import pallas as pl


def kernel(x, prev_embs, W_gc1, b_gc1, W_gc2, b_gc2, W_linear, b_linear, weight_lin, bias_lin, w_q, w_k, w_v, mp_adj, edges):
    raise NotImplementedError("write your pallas kernel here")


# speedup vs baseline: 4.5166x; 4.5166x over previous
import functools

import jax
import jax.numpy as jnp
from jax import lax
from jax.experimental import pallas as pl
from jax.experimental.pallas import tpu as pltpu
from jax.experimental.pallas import tpu_sc as plsc

N = 10000
E = 160000
F_IN = 256
H = 256
S = N // 2

NC = 2
NS = 16
NW = NC * NS
NP = 10240
EP = 163840

_f32 = jnp.float32
_i32 = jnp.int32


def _sc_mesh():
    return plsc.VectorSubcoreMesh(
        core_axis_name="c", subcore_axis_name="s", num_cores=NC, num_subcores=NS
    )


_SC_PARAMS = pltpu.CompilerParams(needs_layout_passes=False)


def _sc_deg(dst_p):
    EPT = EP // NS
    RPT = NP // NS

    @functools.partial(
        pl.kernel,
        mesh=_sc_mesh(),
        compiler_params=_SC_PARAMS,
        out_type=jax.ShapeDtypeStruct((NP,), _f32),
        scratch_types=[
            pltpu.VMEM((NP,), _f32),
            pltpu.VMEM((512,), _i32),
            pltpu.VMEM((NS, RPT), _f32),
            pltpu.VMEM_SHARED((NS, NP), _f32),
        ],
    )
    def k(dst_hbm, out_hbm, hist, dbuf, rbuf, hstage):
        c = lax.axis_index("c")
        s = lax.axis_index("s")

        @pl.when(c == 0)
        def _():
            zero16 = jnp.zeros((16,), _f32)
            one16 = jnp.ones((16,), _f32)

            def zbody(i, carry):
                hist[pl.ds(i * 16, 16)] = zero16
                return carry

            lax.fori_loop(0, NP // 16, zbody, 0)

            ebase = s * EPT

            def chunk(j, carry):
                pltpu.sync_copy(dst_hbm.at[pl.ds(ebase + j * 512, 512)], dbuf)

                def inner(t, c2):
                    idx = dbuf[pl.ds(t * 16, 16)]
                    plsc.addupdate_scatter(hist, [idx], one16)
                    return c2

                lax.fori_loop(0, 32, inner, 0)
                return carry

            lax.fori_loop(0, EPT // 512, chunk, 0)

            pltpu.sync_copy(hist, hstage.at[s])
            plsc.subcore_barrier()

            pltpu.sync_copy(hstage.at[:, pl.ds(s * RPT, RPT)], rbuf)

            def red(kk, carry):
                v = jnp.zeros((16,), _f32)
                for r in range(NS):
                    v = v + rbuf[r, pl.ds(kk * 16, 16)]
                hist[pl.ds(kk * 16, 16)] = v
                return carry

            lax.fori_loop(0, RPT // 16, red, 0)
            pltpu.sync_copy(hist.at[pl.ds(0, RPT)], out_hbm.at[pl.ds(s * RPT, RPT)])

    return k(dst_p)


def _sc_agg(hlo, hhi, src_p, dst_p):
    EPT = EP // NS
    NCH = EPT // 128
    RPT = NP // NS

    @functools.partial(
        pl.kernel,
        mesh=_sc_mesh(),
        compiler_params=_SC_PARAMS,
        out_type=(
            jax.ShapeDtypeStruct((NP, 128), _f32),
            jax.ShapeDtypeStruct((NP, 128), _f32),
        ),
        scratch_types=[
            pltpu.VMEM_SHARED((NP, 128), _f32),
            pltpu.VMEM((128, 128), _f32),
            pltpu.VMEM((128,), _i32),
            pltpu.VMEM((128,), _i32),
            pltpu.SemaphoreType.DMA,
        ],
    )
    def k(hlo_hbm, hhi_hbm, src_hbm, dst_hbm, outlo_hbm, outhi_hbm,
          acc, rows, sidx, didx, sem):
        c = lax.axis_index("c")
        s = lax.axis_index("s")
        rbase = s * RPT

        @pl.when(c == 0)
        def _():
            pltpu.sync_copy(hlo_hbm.at[pl.ds(rbase, RPT)], acc.at[pl.ds(rbase, RPT)])

        @pl.when(c == 1)
        def _():
            pltpu.sync_copy(hhi_hbm.at[pl.ds(rbase, RPT)], acc.at[pl.ds(rbase, RPT)])

        plsc.subcore_barrier()

        ebase = s * EPT

        def chunk(j, carry):
            off = ebase + j * 128
            pltpu.sync_copy(src_hbm.at[pl.ds(off, 128)], sidx)
            pltpu.sync_copy(dst_hbm.at[pl.ds(off, 128)], didx)

            @pl.when(c == 0)
            def _():
                pltpu.async_copy(hlo_hbm.at[sidx], rows, sem).wait()

            @pl.when(c == 1)
            def _():
                pltpu.async_copy(hhi_hbm.at[sidx], rows, sem).wait()

            pltpu.sync_copy(rows, acc.at[didx], add=True)
            return carry

        lax.fori_loop(0, NCH, chunk, 0)
        plsc.subcore_barrier()

        @pl.when(c == 0)
        def _():
            pltpu.sync_copy(acc.at[pl.ds(rbase, RPT)], outlo_hbm.at[pl.ds(rbase, RPT)])

        @pl.when(c == 1)
        def _():
            pltpu.sync_copy(acc.at[pl.ds(rbase, RPT)], outhi_hbm.at[pl.ds(rbase, RPT)])

    return k(hlo, hhi, src_p, dst_p)


def _sc_edge(g_tab, f_tab, e0_p, e1_p, bias_lin):
    EPW = EP // NW
    NCH = EPW // 128

    @functools.partial(
        pl.kernel,
        mesh=_sc_mesh(),
        compiler_params=_SC_PARAMS,
        out_type=jax.ShapeDtypeStruct((EP,), _f32),
        scratch_types=[
            pltpu.VMEM((128, 256), _f32),
            pltpu.VMEM((128, 256), _f32),
            pltpu.VMEM((128,), _i32),
            pltpu.VMEM((128,), _i32),
            pltpu.VMEM((EPW,), _f32),
            pltpu.VMEM((256,), _f32),
            pltpu.SemaphoreType.DMA,
            pltpu.SemaphoreType.DMA,
        ],
    )
    def k(g_hbm, f_hbm, e0_hbm, e1_hbm, bias_hbm, out_hbm,
          gr, fr, i0, i1, ob, bb, sem0, sem1):
        c = lax.axis_index("c")
        s = lax.axis_index("s")
        wid = s * NC + c
        base = wid * EPW

        pltpu.sync_copy(bias_hbm, bb)
        bacc = jnp.zeros((16,), _f32)
        for t in range(16):
            bacc = bacc + bb[pl.ds(t * 16, 16)]
        bsum = jnp.sum(bacc)
        bvec = jnp.full((16,), 1.0, _f32) * bsum

        lane = lax.broadcasted_iota(_i32, (16,), 0)

        def chunk(j, carry):
            off = base + j * 128
            pltpu.sync_copy(e0_hbm.at[pl.ds(off, 128)], i0)
            pltpu.sync_copy(e1_hbm.at[pl.ds(off, 128)], i1)
            cp0 = pltpu.async_copy(g_hbm.at[i0], gr, sem0)
            cp1 = pltpu.async_copy(f_hbm.at[i1], fr, sem1)
            cp0.wait()
            cp1.wait()

            def group(g16, c2):
                v = jnp.zeros((16,), _f32)
                for e16 in range(16):
                    e = g16 * 16 + e16
                    acc = jnp.zeros((16,), _f32)
                    for t in range(16):
                        acc = acc + gr[e, pl.ds(t * 16, 16)] * fr[e, pl.ds(t * 16, 16)]
                    v = jnp.where(lane == e16, jnp.sum(acc), v)
                ob[pl.ds(j * 128 + g16 * 16, 16)] = v
                return c2

            lax.fori_loop(0, 8, group, 0)
            return carry

        lax.fori_loop(0, NCH, chunk, 0)

        def sig(i, carry):
            v = ob[pl.ds(i * 16, 16)] + bvec
            ob[pl.ds(i * 16, 16)] = 1.0 / (1.0 + jnp.exp(-v))
            return carry

        lax.fori_loop(0, EPW // 16, sig, 0)
        pltpu.sync_copy(ob, out_hbm.at[pl.ds(base, EPW)])

    return k(g_tab, f_tab, e0_p, e1_p, bias_lin)


def _tc1(x_p, W1, deg_col):
    R = NP // 1024

    def body(x_ref, w_ref, deg_ref, lo_ref, hi_ref, dinv_ref):
        dinv = lax.rsqrt(deg_ref[...] + 1.0)
        h = jnp.dot(x_ref[...], w_ref[...], preferred_element_type=_f32) * dinv
        lo_ref[...] = h[:, :128]
        hi_ref[...] = h[:, 128:]
        dinv_ref[...] = dinv

    return pl.pallas_call(
        body,
        grid=(R,),
        in_specs=[
            pl.BlockSpec((1024, F_IN), lambda r: (r, 0)),
            pl.BlockSpec((F_IN, H), lambda r: (0, 0)),
            pl.BlockSpec((1024, 1), lambda r: (r, 0)),
        ],
        out_specs=[
            pl.BlockSpec((1024, 128), lambda r: (r, 0)),
            pl.BlockSpec((1024, 128), lambda r: (r, 0)),
            pl.BlockSpec((1024, 1), lambda r: (r, 0)),
        ],
        out_shape=[
            jax.ShapeDtypeStruct((NP, 128), _f32),
            jax.ShapeDtypeStruct((NP, 128), _f32),
            jax.ShapeDtypeStruct((NP, 1), _f32),
        ],
    )(x_p, W1, deg_col)


def _tc2(a1lo, a1hi, dinv, b1r, W2):
    R = NP // 1024

    def body(lo_ref, hi_ref, dinv_ref, b_ref, w_ref, olo_ref, ohi_ref):
        dinv = dinv_ref[...]
        agg = jnp.concatenate([lo_ref[...], hi_ref[...]], axis=1)
        h = jnp.tanh(agg * dinv + b_ref[...])
        h2 = jnp.dot(h, w_ref[...], preferred_element_type=_f32) * dinv
        olo_ref[...] = h2[:, :128]
        ohi_ref[...] = h2[:, 128:]

    return pl.pallas_call(
        body,
        grid=(R,),
        in_specs=[
            pl.BlockSpec((1024, 128), lambda r: (r, 0)),
            pl.BlockSpec((1024, 128), lambda r: (r, 0)),
            pl.BlockSpec((1024, 1), lambda r: (r, 0)),
            pl.BlockSpec((1, H), lambda r: (0, 0)),
            pl.BlockSpec((H, H), lambda r: (0, 0)),
        ],
        out_specs=[
            pl.BlockSpec((1024, 128), lambda r: (r, 0)),
            pl.BlockSpec((1024, 128), lambda r: (r, 0)),
        ],
        out_shape=[
            jax.ShapeDtypeStruct((NP, 128), _f32),
            jax.ShapeDtypeStruct((NP, 128), _f32),
        ],
    )(a1lo, a1hi, dinv, b1r, W2)


def _tc3(a2lo, a2hi, dinv, b2r, prev_p, w_q, w_k, w_v, Wl1, Wl2, blr, wl, wlT):
    R = NP // 1024

    def body(lo_ref, hi_ref, dinv_ref, b_ref, prev_ref, wq_ref, wk_ref,
             wv_ref, wl1_ref, wl2_ref, bl_ref, wlin_ref, wlinT_ref,
             g_ref, f_ref):
        r = pl.program_id(0)
        dinv = dinv_ref[...]
        agg = jnp.concatenate([lo_ref[...], hi_ref[...]], axis=1)
        emb = jnp.tanh(agg * dinv + b_ref[...])
        prev = prev_ref[...]

        q = jnp.dot(emb, wq_ref[...], preferred_element_type=_f32)
        ke = jnp.dot(emb, wk_ref[...], preferred_element_type=_f32)
        ve = jnp.dot(emb, wv_ref[...], preferred_element_type=_f32)
        kp = jnp.dot(prev, wk_ref[...], preferred_element_type=_f32)
        vp = jnp.dot(prev, wv_ref[...], preferred_element_type=_f32)

        a0 = jnp.sum(q * kp, axis=1, keepdims=True) * (1.0 / 16.0)
        a1 = jnp.sum(q * ke, axis=1, keepdims=True) * (1.0 / 16.0)
        m = jnp.maximum(a0, a1)
        e0 = jnp.exp(a0 - m)
        e1 = jnp.exp(a1 - m)
        attn = (e0 * vp + e1 * ve) / (e0 + e1)

        rowid = r * 1024 + lax.broadcasted_iota(_i32, (1024, 1), 0)
        ae = jnp.where(rowid < S, attn, ve)

        final = jnp.tanh(
            jnp.dot(emb, wl1_ref[...], preferred_element_type=_f32)
            + jnp.dot(ae, wl2_ref[...], preferred_element_type=_f32)
            + bl_ref[...]
        )
        sym = (wlin_ref[...] + wlinT_ref[...]) * 0.5
        g_ref[...] = jnp.dot(final, sym, preferred_element_type=_f32)
        f_ref[...] = final

    whole = lambda shape: pl.BlockSpec(shape, lambda r: tuple(0 for _ in shape))
    return pl.pallas_call(
        body,
        grid=(R,),
        in_specs=[
            pl.BlockSpec((1024, 128), lambda r: (r, 0)),
            pl.BlockSpec((1024, 128), lambda r: (r, 0)),
            pl.BlockSpec((1024, 1), lambda r: (r, 0)),
            whole((1, H)),
            pl.BlockSpec((1024, H), lambda r: (r, 0)),
            whole((H, H)),
            whole((H, H)),
            whole((H, H)),
            whole((H, H)),
            whole((H, H)),
            whole((1, H)),
            whole((H, H)),
            whole((H, H)),
        ],
        out_specs=[
            pl.BlockSpec((1024, H), lambda r: (r, 0)),
            pl.BlockSpec((1024, H), lambda r: (r, 0)),
        ],
        out_shape=[
            jax.ShapeDtypeStruct((NP, H), _f32),
            jax.ShapeDtypeStruct((NP, H), _f32),
        ],
    )(a2lo, a2hi, dinv, b2r, prev_p, w_q, w_k, w_v, Wl1, Wl2, blr, wl, wlT)


def kernel(x, prev_embs, W_gc1, b_gc1, W_gc2, b_gc2, W_linear, b_linear,
           weight_lin, bias_lin, w_q, w_k, w_v, mp_adj, edges):
    src = mp_adj[0].astype(_i32)
    dst = mp_adj[1].astype(_i32)
    e0 = edges[:, 0].astype(_i32)
    e1 = edges[:, 1].astype(_i32)

    pad_e = EP - E
    src_p = jnp.concatenate([src, jnp.zeros((pad_e,), _i32)])
    dst_p = jnp.concatenate([dst, jnp.full((pad_e,), N, _i32)])
    e0_p = jnp.concatenate([e0, jnp.zeros((pad_e,), _i32)])
    e1_p = jnp.concatenate([e1, jnp.zeros((pad_e,), _i32)])

    x_p = jnp.concatenate([x, jnp.zeros((NP - N, F_IN), _f32)])
    prev_p = jnp.concatenate([prev_embs[0], jnp.zeros((NP - S, H), _f32)])

    deg = _sc_deg(dst_p)
    deg_col = deg.reshape(NP, 1)

    h1lo, h1hi, dinv = _tc1(x_p, W_gc1, deg_col)
    a1lo, a1hi = _sc_agg(h1lo, h1hi, src_p, dst_p)
    h2lo, h2hi = _tc2(a1lo, a1hi, dinv, b_gc1.reshape(1, H), W_gc2)
    a2lo, a2hi = _sc_agg(h2lo, h2hi, src_p, dst_p)
    g_tab, f_tab = _tc3(
        a2lo, a2hi, dinv, b_gc2.reshape(1, H), prev_p, w_q, w_k, w_v,
        W_linear[:H], W_linear[H:], b_linear.reshape(1, H),
        weight_lin, weight_lin.T,
    )
    out_p = _sc_edge(g_tab, f_tab, e0_p, e1_p, bias_lin)
    return out_p[:E]


# speedup vs baseline: 5.8548x; 1.2963x over previous
import functools

import jax
import jax.numpy as jnp
from jax import lax
from jax.experimental import pallas as pl
from jax.experimental.pallas import tpu as pltpu
from jax.experimental.pallas import tpu_sc as plsc

N = 10000
E = 160000
F_IN = 256
H = 256
S = N // 2

NC = 2
NS = 16
NW = NC * NS
NP = 10240
EP = 163840

_f32 = jnp.float32
_i32 = jnp.int32


def _sc_mesh():
    return plsc.VectorSubcoreMesh(
        core_axis_name="c", subcore_axis_name="s", num_cores=NC, num_subcores=NS
    )


_SC_PARAMS = pltpu.CompilerParams(needs_layout_passes=False)


def _sc_deg(dst_p):
    EPT = EP // NS
    RPT = NP // NS

    @functools.partial(
        pl.kernel,
        mesh=_sc_mesh(),
        compiler_params=_SC_PARAMS,
        out_type=jax.ShapeDtypeStruct((NP,), _f32),
        scratch_types=[
            pltpu.VMEM((NP,), _f32),
            pltpu.VMEM((512,), _i32),
            pltpu.VMEM((NS, RPT), _f32),
            pltpu.VMEM_SHARED((NS, NP), _f32),
        ],
    )
    def k(dst_hbm, out_hbm, hist, dbuf, rbuf, hstage):
        c = lax.axis_index("c")
        s = lax.axis_index("s")

        @pl.when(c == 0)
        def _():
            zero16 = jnp.zeros((16,), _f32)
            one16 = jnp.ones((16,), _f32)

            def zbody(i, carry):
                hist[pl.ds(i * 16, 16)] = zero16
                return carry

            lax.fori_loop(0, NP // 16, zbody, 0)

            ebase = s * EPT

            def chunk(j, carry):
                pltpu.sync_copy(dst_hbm.at[pl.ds(ebase + j * 512, 512)], dbuf)

                def inner(t, c2):
                    idx = dbuf[pl.ds(t * 16, 16)]
                    plsc.addupdate_scatter(hist, [idx], one16)
                    return c2

                lax.fori_loop(0, 32, inner, 0)
                return carry

            lax.fori_loop(0, EPT // 512, chunk, 0)

            pltpu.sync_copy(hist, hstage.at[s])
            plsc.subcore_barrier()

            pltpu.sync_copy(hstage.at[:, pl.ds(s * RPT, RPT)], rbuf)

            def red(kk, carry):
                v = jnp.zeros((16,), _f32)
                for r in range(NS):
                    v = v + rbuf[r, pl.ds(kk * 16, 16)]
                hist[pl.ds(kk * 16, 16)] = v
                return carry

            lax.fori_loop(0, RPT // 16, red, 0)
            pltpu.sync_copy(hist.at[pl.ds(0, RPT)], out_hbm.at[pl.ds(s * RPT, RPT)])

    return k(dst_p)


def _sc_agg(hlo, hhi, src_p, dst_p):
    EPT = EP // NS
    CH = 64
    NCH = EPT // CH
    NPH = 4
    CPP = NCH // NPH
    RPT = NP // NS
    NB = 2

    @functools.partial(
        pl.kernel,
        mesh=_sc_mesh(),
        compiler_params=_SC_PARAMS,
        out_type=(
            jax.ShapeDtypeStruct((NP, 128), _f32),
            jax.ShapeDtypeStruct((NP, 128), _f32),
        ),
        scratch_types=[
            pltpu.VMEM_SHARED((NP, 128), _f32),
            [pltpu.VMEM((CH, 128), _f32) for _ in range(NB)],
            pltpu.VMEM((CPP, CH), _i32),
            pltpu.VMEM((CPP, CH), _i32),
            [pltpu.SemaphoreType.DMA for _ in range(NB)],
            [pltpu.SemaphoreType.DMA for _ in range(NB)],
        ],
    )
    def k(hlo_hbm, hhi_hbm, src_hbm, dst_hbm, outlo_hbm, outhi_hbm,
          acc, rows, sidx, didx, gsem, ssem):
        c = lax.axis_index("c")
        s = lax.axis_index("s")
        rbase = s * RPT

        @pl.when(c == 0)
        def _():
            pltpu.sync_copy(hlo_hbm.at[pl.ds(rbase, RPT)], acc.at[pl.ds(rbase, RPT)])

        @pl.when(c == 1)
        def _():
            pltpu.sync_copy(hhi_hbm.at[pl.ds(rbase, RPT)], acc.at[pl.ds(rbase, RPT)])

        plsc.subcore_barrier()

        def start_gather(b, j):
            @pl.when(c == 0)
            def _():
                pltpu.async_copy(hlo_hbm.at[sidx.at[j]], rows[b], gsem[b])

            @pl.when(c == 1)
            def _():
                pltpu.async_copy(hhi_hbm.at[sidx.at[j]], rows[b], gsem[b])

        def wait_gather(b, j):
            pltpu.make_async_copy(hlo_hbm.at[sidx.at[j]], rows[b], gsem[b]).wait()

        def start_scatter(b, j):
            pltpu.async_copy(rows[b], acc.at[didx.at[j]], ssem[b], add=True)

        def wait_scatter(b, j):
            pltpu.make_async_copy(rows[b], acc.at[didx.at[j]], ssem[b]).wait()

        for ph in range(NPH):
            cbase = s * NCH + ph * CPP
            pltpu.sync_copy(src_hbm.at[pl.ds(cbase, CPP)], sidx)
            pltpu.sync_copy(dst_hbm.at[pl.ds(cbase, CPP)], didx)
            for b in range(NB):
                start_gather(b, b)
                wait_gather(b, b)
                start_scatter(b, b)

            def block(jo, carry):
                for b in range(NB):
                    j = jo * NB + b
                    wait_scatter(b, j)
                    start_gather(b, j)
                    wait_gather(b, j)
                    start_scatter(b, j)
                return carry

            lax.fori_loop(1, CPP // NB, block, 0)
            for b in range(NB):
                wait_scatter(b, 0)

        plsc.subcore_barrier()

        @pl.when(c == 0)
        def _():
            pltpu.sync_copy(acc.at[pl.ds(rbase, RPT)], outlo_hbm.at[pl.ds(rbase, RPT)])

        @pl.when(c == 1)
        def _():
            pltpu.sync_copy(acc.at[pl.ds(rbase, RPT)], outhi_hbm.at[pl.ds(rbase, RPT)])

    return k(hlo, hhi, src_p, dst_p)


def _sc_edge(g_tab, f_tab, e0_p, e1_p, bias_lin):
    EPW = EP // NW
    CH = 64
    NCH = EPW // CH

    @functools.partial(
        pl.kernel,
        mesh=_sc_mesh(),
        compiler_params=_SC_PARAMS,
        out_type=jax.ShapeDtypeStruct((EP,), _f32),
        scratch_types=[
            [pltpu.VMEM((CH, 256), _f32) for _ in range(2)],
            [pltpu.VMEM((CH, 256), _f32) for _ in range(2)],
            pltpu.VMEM((NCH, CH), _i32),
            pltpu.VMEM((NCH, CH), _i32),
            pltpu.VMEM((EPW,), _f32),
            pltpu.VMEM((256,), _f32),
            [pltpu.SemaphoreType.DMA for _ in range(2)],
            [pltpu.SemaphoreType.DMA for _ in range(2)],
        ],
    )
    def k(g_hbm, f_hbm, e0_hbm, e1_hbm, bias_hbm, out_hbm,
          gr, fr, i0, i1, ob, bb, gsem, fsem):
        c = lax.axis_index("c")
        s = lax.axis_index("s")
        wid = s * NC + c
        base = wid * EPW

        pltpu.sync_copy(bias_hbm, bb)
        pltpu.sync_copy(e0_hbm.at[pl.ds(wid * NCH, NCH)], i0)
        pltpu.sync_copy(e1_hbm.at[pl.ds(wid * NCH, NCH)], i1)
        bacc = jnp.zeros((16,), _f32)
        for t in range(16):
            bacc = bacc + bb[pl.ds(t * 16, 16)]
        bsum = jnp.sum(bacc)
        bvec = jnp.full((16,), 1.0, _f32) * bsum

        lane = lax.broadcasted_iota(_i32, (16,), 0)

        def start_gathers(b, j):
            pltpu.async_copy(g_hbm.at[i0.at[j]], gr[b], gsem[b])
            pltpu.async_copy(f_hbm.at[i1.at[j]], fr[b], fsem[b])

        def wait_gathers(b, j):
            pltpu.make_async_copy(g_hbm.at[i0.at[j]], gr[b], gsem[b]).wait()
            pltpu.make_async_copy(f_hbm.at[i1.at[j]], fr[b], fsem[b]).wait()

        start_gathers(0, 0)
        start_gathers(1, 1)

        def compute(b, j):
            def group(g16, c2):
                v = jnp.zeros((16,), _f32)
                for e16 in range(16):
                    e = g16 * 16 + e16
                    acc = jnp.zeros((16,), _f32)
                    for t in range(16):
                        acc = acc + gr[b][e, pl.ds(t * 16, 16)] * fr[b][e, pl.ds(t * 16, 16)]
                    v = jnp.where(lane == e16, jnp.sum(acc), v)
                ob[pl.ds(j * CH + g16 * 16, 16)] = v
                return c2

            lax.fori_loop(0, CH // 16, group, 0)

        def block(jo, carry):
            for b in range(2):
                j = jo * 2 + b
                wait_gathers(b, j)
                compute(b, j)
                nxt = jnp.minimum(j + 2, NCH - 1)
                start_gathers(b, nxt)
            return carry

        lax.fori_loop(0, NCH // 2, block, 0)
        wait_gathers(0, 0)
        wait_gathers(1, 0)

        def sig(i, carry):
            v = ob[pl.ds(i * 16, 16)] + bvec
            ob[pl.ds(i * 16, 16)] = 1.0 / (1.0 + jnp.exp(-v))
            return carry

        lax.fori_loop(0, EPW // 16, sig, 0)
        pltpu.sync_copy(ob, out_hbm.at[pl.ds(base, EPW)])

    return k(g_tab, f_tab, e0_p, e1_p, bias_lin)


def _tc1(x_p, W1, deg_col):
    R = NP // 1024

    def body(x_ref, w_ref, deg_ref, lo_ref, hi_ref, dinv_ref):
        dinv = lax.rsqrt(deg_ref[...] + 1.0)
        h = jnp.dot(x_ref[...], w_ref[...], preferred_element_type=_f32) * dinv
        lo_ref[...] = h[:, :128]
        hi_ref[...] = h[:, 128:]
        dinv_ref[...] = dinv

    return pl.pallas_call(
        body,
        grid=(R,),
        in_specs=[
            pl.BlockSpec((1024, F_IN), lambda r: (r, 0)),
            pl.BlockSpec((F_IN, H), lambda r: (0, 0)),
            pl.BlockSpec((1024, 1), lambda r: (r, 0)),
        ],
        out_specs=[
            pl.BlockSpec((1024, 128), lambda r: (r, 0)),
            pl.BlockSpec((1024, 128), lambda r: (r, 0)),
            pl.BlockSpec((1024, 1), lambda r: (r, 0)),
        ],
        out_shape=[
            jax.ShapeDtypeStruct((NP, 128), _f32),
            jax.ShapeDtypeStruct((NP, 128), _f32),
            jax.ShapeDtypeStruct((NP, 1), _f32),
        ],
    )(x_p, W1, deg_col)


def _tc2(a1lo, a1hi, dinv, b1r, W2):
    R = NP // 1024

    def body(lo_ref, hi_ref, dinv_ref, b_ref, w_ref, olo_ref, ohi_ref):
        dinv = dinv_ref[...]
        agg = jnp.concatenate([lo_ref[...], hi_ref[...]], axis=1)
        h = jnp.tanh(agg * dinv + b_ref[...])
        h2 = jnp.dot(h, w_ref[...], preferred_element_type=_f32) * dinv
        olo_ref[...] = h2[:, :128]
        ohi_ref[...] = h2[:, 128:]

    return pl.pallas_call(
        body,
        grid=(R,),
        in_specs=[
            pl.BlockSpec((1024, 128), lambda r: (r, 0)),
            pl.BlockSpec((1024, 128), lambda r: (r, 0)),
            pl.BlockSpec((1024, 1), lambda r: (r, 0)),
            pl.BlockSpec((1, H), lambda r: (0, 0)),
            pl.BlockSpec((H, H), lambda r: (0, 0)),
        ],
        out_specs=[
            pl.BlockSpec((1024, 128), lambda r: (r, 0)),
            pl.BlockSpec((1024, 128), lambda r: (r, 0)),
        ],
        out_shape=[
            jax.ShapeDtypeStruct((NP, 128), _f32),
            jax.ShapeDtypeStruct((NP, 128), _f32),
        ],
    )(a1lo, a1hi, dinv, b1r, W2)


def _tc3(a2lo, a2hi, dinv, b2r, prev_p, w_q, w_k, w_v, Wl1, Wl2, blr, wl, wlT):
    R = NP // 1024

    def body(lo_ref, hi_ref, dinv_ref, b_ref, prev_ref, wq_ref, wk_ref,
             wv_ref, wl1_ref, wl2_ref, bl_ref, wlin_ref, wlinT_ref,
             g_ref, f_ref):
        r = pl.program_id(0)
        dinv = dinv_ref[...]
        agg = jnp.concatenate([lo_ref[...], hi_ref[...]], axis=1)
        emb = jnp.tanh(agg * dinv + b_ref[...])
        prev = prev_ref[...]

        q = jnp.dot(emb, wq_ref[...], preferred_element_type=_f32)
        ke = jnp.dot(emb, wk_ref[...], preferred_element_type=_f32)
        ve = jnp.dot(emb, wv_ref[...], preferred_element_type=_f32)
        kp = jnp.dot(prev, wk_ref[...], preferred_element_type=_f32)
        vp = jnp.dot(prev, wv_ref[...], preferred_element_type=_f32)

        a0 = jnp.sum(q * kp, axis=1, keepdims=True) * (1.0 / 16.0)
        a1 = jnp.sum(q * ke, axis=1, keepdims=True) * (1.0 / 16.0)
        m = jnp.maximum(a0, a1)
        e0 = jnp.exp(a0 - m)
        e1 = jnp.exp(a1 - m)
        attn = (e0 * vp + e1 * ve) / (e0 + e1)

        rowid = r * 1024 + lax.broadcasted_iota(_i32, (1024, 1), 0)
        ae = jnp.where(rowid < S, attn, ve)

        final = jnp.tanh(
            jnp.dot(emb, wl1_ref[...], preferred_element_type=_f32)
            + jnp.dot(ae, wl2_ref[...], preferred_element_type=_f32)
            + bl_ref[...]
        )
        sym = (wlin_ref[...] + wlinT_ref[...]) * 0.5
        g_ref[...] = jnp.dot(final, sym, preferred_element_type=_f32)
        f_ref[...] = final

    whole = lambda shape: pl.BlockSpec(shape, lambda r: tuple(0 for _ in shape))
    return pl.pallas_call(
        body,
        grid=(R,),
        in_specs=[
            pl.BlockSpec((1024, 128), lambda r: (r, 0)),
            pl.BlockSpec((1024, 128), lambda r: (r, 0)),
            pl.BlockSpec((1024, 1), lambda r: (r, 0)),
            whole((1, H)),
            pl.BlockSpec((1024, H), lambda r: (r, 0)),
            whole((H, H)),
            whole((H, H)),
            whole((H, H)),
            whole((H, H)),
            whole((H, H)),
            whole((1, H)),
            whole((H, H)),
            whole((H, H)),
        ],
        out_specs=[
            pl.BlockSpec((1024, H), lambda r: (r, 0)),
            pl.BlockSpec((1024, H), lambda r: (r, 0)),
        ],
        out_shape=[
            jax.ShapeDtypeStruct((NP, H), _f32),
            jax.ShapeDtypeStruct((NP, H), _f32),
        ],
    )(a2lo, a2hi, dinv, b2r, prev_p, w_q, w_k, w_v, Wl1, Wl2, blr, wl, wlT)


def kernel(x, prev_embs, W_gc1, b_gc1, W_gc2, b_gc2, W_linear, b_linear,
           weight_lin, bias_lin, w_q, w_k, w_v, mp_adj, edges):
    src = mp_adj[0].astype(_i32)
    dst = mp_adj[1].astype(_i32)
    e0 = edges[:, 0].astype(_i32)
    e1 = edges[:, 1].astype(_i32)

    pad_e = EP - E
    src_p = jnp.concatenate([src, jnp.zeros((pad_e,), _i32)])
    dst_p = jnp.concatenate([dst, jnp.full((pad_e,), N, _i32)])
    e0_p = jnp.concatenate([e0, jnp.zeros((pad_e,), _i32)])
    e1_p = jnp.concatenate([e1, jnp.zeros((pad_e,), _i32)])

    x_p = jnp.concatenate([x, jnp.zeros((NP - N, F_IN), _f32)])
    prev_p = jnp.concatenate([prev_embs[0], jnp.zeros((NP - S, H), _f32)])

    deg = _sc_deg(dst_p)
    deg_col = deg.reshape(NP, 1)
    src2d = src_p.reshape(EP // 64, 64)
    dst2d = dst_p.reshape(EP // 64, 64)

    h1lo, h1hi, dinv = _tc1(x_p, W_gc1, deg_col)
    a1lo, a1hi = _sc_agg(h1lo, h1hi, src2d, dst2d)
    h2lo, h2hi = _tc2(a1lo, a1hi, dinv, b_gc1.reshape(1, H), W_gc2)
    a2lo, a2hi = _sc_agg(h2lo, h2hi, src2d, dst2d)
    g_tab, f_tab = _tc3(
        a2lo, a2hi, dinv, b_gc2.reshape(1, H), prev_p, w_q, w_k, w_v,
        W_linear[:H], W_linear[H:], b_linear.reshape(1, H),
        weight_lin, weight_lin.T,
    )
    out_p = _sc_edge(g_tab, f_tab, e0_p.reshape(EP // 64, 64),
                     e1_p.reshape(EP // 64, 64), bias_lin)
    return out_p[:E]


# speedup vs baseline: 6.2795x; 1.0725x over previous
import functools

import jax
import jax.numpy as jnp
from jax import lax
from jax.experimental import pallas as pl
from jax.experimental.pallas import tpu as pltpu
from jax.experimental.pallas import tpu_sc as plsc

N = 10000
E = 160000
F_IN = 256
H = 256
S = N // 2

NC = 2
NS = 16
NW = NC * NS
NP = 10240
EP = 163840

_f32 = jnp.float32
_i32 = jnp.int32


def _sc_mesh():
    return plsc.VectorSubcoreMesh(
        core_axis_name="c", subcore_axis_name="s", num_cores=NC, num_subcores=NS
    )


_SC_PARAMS = pltpu.CompilerParams(needs_layout_passes=False)


def _sc_deg(dst_p):
    EPT = EP // NS
    RPT = NP // NS

    @functools.partial(
        pl.kernel,
        mesh=_sc_mesh(),
        compiler_params=_SC_PARAMS,
        out_type=jax.ShapeDtypeStruct((NP,), _f32),
        scratch_types=[
            pltpu.VMEM((NP,), _f32),
            pltpu.VMEM((512,), _i32),
            pltpu.VMEM((NS, RPT), _f32),
            pltpu.VMEM_SHARED((NS, NP), _f32),
        ],
    )
    def k(dst_hbm, out_hbm, hist, dbuf, rbuf, hstage):
        c = lax.axis_index("c")
        s = lax.axis_index("s")

        @pl.when(c == 0)
        def _():
            zero16 = jnp.zeros((16,), _f32)
            one16 = jnp.ones((16,), _f32)

            def zbody(i, carry):
                hist[pl.ds(i * 16, 16)] = zero16
                return carry

            lax.fori_loop(0, NP // 16, zbody, 0)

            ebase = s * EPT

            def chunk(j, carry):
                pltpu.sync_copy(dst_hbm.at[pl.ds(ebase + j * 512, 512)], dbuf)

                def inner(t, c2):
                    idx = dbuf[pl.ds(t * 16, 16)]
                    plsc.addupdate_scatter(hist, [idx], one16)
                    return c2

                lax.fori_loop(0, 32, inner, 0)
                return carry

            lax.fori_loop(0, EPT // 512, chunk, 0)

            pltpu.sync_copy(hist, hstage.at[s])
            plsc.subcore_barrier()

            pltpu.sync_copy(hstage.at[:, pl.ds(s * RPT, RPT)], rbuf)

            def red(kk, carry):
                v = jnp.zeros((16,), _f32)
                for r in range(NS):
                    v = v + rbuf[r, pl.ds(kk * 16, 16)]
                hist[pl.ds(kk * 16, 16)] = v
                return carry

            lax.fori_loop(0, RPT // 16, red, 0)
            pltpu.sync_copy(hist.at[pl.ds(0, RPT)], out_hbm.at[pl.ds(s * RPT, RPT)])

    return k(dst_p)


def _sc_agg(hlo, hhi, src_hbm2d, dst_hbm2d):
    EPT = EP // NS
    CH = 128
    NCH = EPT // CH
    NPH = 5
    CPP = NCH // NPH
    RPT = NP // NS
    NB = 2

    @functools.partial(
        pl.kernel,
        mesh=_sc_mesh(),
        compiler_params=_SC_PARAMS,
        out_type=(
            jax.ShapeDtypeStruct((NP, 128), _f32),
            jax.ShapeDtypeStruct((NP, 128), _f32),
        ),
        scratch_types=[
            pltpu.VMEM_SHARED((NP, 128), _f32),
            [pltpu.VMEM((CH, 128), _f32) for _ in range(NB)],
            pltpu.VMEM((CPP, CH), _i32),
            pltpu.VMEM((CPP, CH), _i32),
            [pltpu.SemaphoreType.DMA for _ in range(NB)],
            [pltpu.SemaphoreType.DMA for _ in range(NB)],
        ],
    )
    def k(hlo_hbm, hhi_hbm, src_hbm, dst_hbm, outlo_hbm, outhi_hbm,
          acc, rows, sidx, didx, gsem, ssem):
        c = lax.axis_index("c")
        s = lax.axis_index("s")
        rbase = s * RPT

        @pl.when(c == 0)
        def _():
            pltpu.sync_copy(hlo_hbm.at[pl.ds(rbase, RPT)], acc.at[pl.ds(rbase, RPT)])

        @pl.when(c == 1)
        def _():
            pltpu.sync_copy(hhi_hbm.at[pl.ds(rbase, RPT)], acc.at[pl.ds(rbase, RPT)])

        plsc.subcore_barrier()

        def start_gather(b, j):
            @pl.when(c == 0)
            def _():
                pltpu.async_copy(hlo_hbm.at[sidx.at[j]], rows[b], gsem[b])

            @pl.when(c == 1)
            def _():
                pltpu.async_copy(hhi_hbm.at[sidx.at[j]], rows[b], gsem[b])

        def wait_gather(b, j):
            pltpu.make_async_copy(hlo_hbm.at[sidx.at[j]], rows[b], gsem[b]).wait()

        def start_scatter(b, j):
            pltpu.async_copy(rows[b], acc.at[didx.at[j]], ssem[b], add=True)

        def wait_scatter(b, j):
            pltpu.make_async_copy(rows[b], acc.at[didx.at[j]], ssem[b]).wait()

        for ph in range(NPH):
            cbase = s * NCH + ph * CPP
            pltpu.sync_copy(src_hbm.at[pl.ds(cbase, CPP)], sidx)
            pltpu.sync_copy(dst_hbm.at[pl.ds(cbase, CPP)], didx)
            for b in range(NB):
                start_gather(b, b)
                wait_gather(b, b)
                start_scatter(b, b)

            def block(jo, carry):
                for b in range(NB):
                    j = jo * NB + b
                    wait_scatter(b, j)
                    start_gather(b, j)
                    wait_gather(b, j)
                    start_scatter(b, j)
                return carry

            lax.fori_loop(1, CPP // NB, block, 0)
            for b in range(NB):
                wait_scatter(b, 0)

        plsc.subcore_barrier()

        @pl.when(c == 0)
        def _():
            pltpu.sync_copy(acc.at[pl.ds(rbase, RPT)], outlo_hbm.at[pl.ds(rbase, RPT)])

        @pl.when(c == 1)
        def _():
            pltpu.sync_copy(acc.at[pl.ds(rbase, RPT)], outhi_hbm.at[pl.ds(rbase, RPT)])

    return k(hlo, hhi, src_hbm2d, dst_hbm2d)


def _sc_edge(g_tab, f_tab, e0r, e1r, bias_lin):
    EPW = EP // NW
    CH = 80
    NCH = EPW // CH

    @functools.partial(
        pl.kernel,
        mesh=_sc_mesh(),
        compiler_params=_SC_PARAMS,
        out_type=jax.ShapeDtypeStruct((EP,), _f32),
        scratch_types=[
            [pltpu.VMEM((CH, 256), _f32) for _ in range(2)],
            [pltpu.VMEM((CH, 256), _f32) for _ in range(2)],
            pltpu.VMEM((NCH, CH), _i32),
            pltpu.VMEM((NCH, CH), _i32),
            pltpu.VMEM((EPW,), _f32),
            pltpu.VMEM((256,), _f32),
            [pltpu.SemaphoreType.DMA for _ in range(2)],
            [pltpu.SemaphoreType.DMA for _ in range(2)],
        ],
    )
    def k(g_hbm, f_hbm, e0_hbm, e1_hbm, bias_hbm, out_hbm,
          gr, fr, i0, i1, ob, bb, gsem, fsem):
        c = lax.axis_index("c")
        s = lax.axis_index("s")
        wid = s * NC + c
        base = wid * EPW

        pltpu.sync_copy(bias_hbm, bb)
        pltpu.sync_copy(e0_hbm.at[pl.ds(wid * NCH, NCH)], i0)
        pltpu.sync_copy(e1_hbm.at[pl.ds(wid * NCH, NCH)], i1)
        bacc = jnp.zeros((16,), _f32)
        for t in range(16):
            bacc = bacc + bb[pl.ds(t * 16, 16)]
        bsum = jnp.sum(bacc)
        bvec = jnp.full((16,), 1.0, _f32) * bsum

        lane = lax.broadcasted_iota(_i32, (16,), 0)

        def start_gathers(b, j):
            pltpu.async_copy(g_hbm.at[i0.at[j]], gr[b], gsem[b])
            pltpu.async_copy(f_hbm.at[i1.at[j]], fr[b], fsem[b])

        def wait_gathers(b, j):
            pltpu.make_async_copy(g_hbm.at[i0.at[j]], gr[b], gsem[b]).wait()
            pltpu.make_async_copy(f_hbm.at[i1.at[j]], fr[b], fsem[b]).wait()

        start_gathers(0, 0)
        start_gathers(1, 1)

        def compute(b, j):
            def group(g16, c2):
                v = jnp.zeros((16,), _f32)
                for e16 in range(16):
                    e = g16 * 16 + e16
                    acc = jnp.zeros((16,), _f32)
                    for t in range(16):
                        acc = acc + gr[b][e, pl.ds(t * 16, 16)] * fr[b][e, pl.ds(t * 16, 16)]
                    v = jnp.where(lane == e16, jnp.sum(acc), v)
                ob[pl.ds(j * CH + g16 * 16, 16)] = v
                return c2

            lax.fori_loop(0, CH // 16, group, 0)

        def block(jo, carry):
            for b in range(2):
                j = jo * 2 + b
                wait_gathers(b, j)
                compute(b, j)
                nxt = jnp.minimum(j + 2, NCH - 1)
                start_gathers(b, nxt)
            return carry

        lax.fori_loop(0, NCH // 2, block, 0)
        wait_gathers(0, 0)
        wait_gathers(1, 0)

        def sig(i, carry):
            v = ob[pl.ds(i * 16, 16)] + bvec
            ob[pl.ds(i * 16, 16)] = 1.0 / (1.0 + jnp.exp(-v))
            return carry

        lax.fori_loop(0, EPW // 16, sig, 0)
        pltpu.sync_copy(ob, out_hbm.at[pl.ds(base, EPW)])

    return k(g_tab, f_tab, e0r, e1r, bias_lin)


def _tc1(x_p, W1, deg_col):
    R = NP // 1024

    def body(x_ref, w_ref, deg_ref, lo_ref, hi_ref, dinv_ref):
        dinv = lax.rsqrt(deg_ref[...] + 1.0)
        h = jnp.dot(x_ref[...], w_ref[...], preferred_element_type=_f32) * dinv
        lo_ref[...] = h[:, :128]
        hi_ref[...] = h[:, 128:]
        dinv_ref[...] = dinv

    return pl.pallas_call(
        body,
        grid=(R,),
        in_specs=[
            pl.BlockSpec((1024, F_IN), lambda r: (r, 0)),
            pl.BlockSpec((F_IN, H), lambda r: (0, 0)),
            pl.BlockSpec((1024, 1), lambda r: (r, 0)),
        ],
        out_specs=[
            pl.BlockSpec((1024, 128), lambda r: (r, 0)),
            pl.BlockSpec((1024, 128), lambda r: (r, 0)),
            pl.BlockSpec((1024, 1), lambda r: (r, 0)),
        ],
        out_shape=[
            jax.ShapeDtypeStruct((NP, 128), _f32),
            jax.ShapeDtypeStruct((NP, 128), _f32),
            jax.ShapeDtypeStruct((NP, 1), _f32),
        ],
    )(x_p, W1, deg_col)


def _tc2(a1lo, a1hi, dinv, b1r, W2):
    R = NP // 1024

    def body(lo_ref, hi_ref, dinv_ref, b_ref, w_ref, olo_ref, ohi_ref):
        dinv = dinv_ref[...]
        agg = jnp.concatenate([lo_ref[...], hi_ref[...]], axis=1)
        h = jnp.tanh(agg * dinv + b_ref[...])
        h2 = jnp.dot(h, w_ref[...], preferred_element_type=_f32) * dinv
        olo_ref[...] = h2[:, :128]
        ohi_ref[...] = h2[:, 128:]

    return pl.pallas_call(
        body,
        grid=(R,),
        in_specs=[
            pl.BlockSpec((1024, 128), lambda r: (r, 0)),
            pl.BlockSpec((1024, 128), lambda r: (r, 0)),
            pl.BlockSpec((1024, 1), lambda r: (r, 0)),
            pl.BlockSpec((1, H), lambda r: (0, 0)),
            pl.BlockSpec((H, H), lambda r: (0, 0)),
        ],
        out_specs=[
            pl.BlockSpec((1024, 128), lambda r: (r, 0)),
            pl.BlockSpec((1024, 128), lambda r: (r, 0)),
        ],
        out_shape=[
            jax.ShapeDtypeStruct((NP, 128), _f32),
            jax.ShapeDtypeStruct((NP, 128), _f32),
        ],
    )(a1lo, a1hi, dinv, b1r, W2)


def _tc3(a2lo, a2hi, dinv, b2r, prev_p, w_q, w_k, w_v, Wl1, Wl2, blr, wl, wlT):
    R = NP // 1024

    def body(lo_ref, hi_ref, dinv_ref, b_ref, prev_ref, wq_ref, wk_ref,
             wv_ref, wl1_ref, wl2_ref, bl_ref, wlin_ref, wlinT_ref,
             g_ref, f_ref):
        r = pl.program_id(0)
        dinv = dinv_ref[...]
        agg = jnp.concatenate([lo_ref[...], hi_ref[...]], axis=1)
        emb = jnp.tanh(agg * dinv + b_ref[...])
        prev = prev_ref[...]

        q = jnp.dot(emb, wq_ref[...], preferred_element_type=_f32)
        ke = jnp.dot(emb, wk_ref[...], preferred_element_type=_f32)
        ve = jnp.dot(emb, wv_ref[...], preferred_element_type=_f32)
        kp = jnp.dot(prev, wk_ref[...], preferred_element_type=_f32)
        vp = jnp.dot(prev, wv_ref[...], preferred_element_type=_f32)

        a0 = jnp.sum(q * kp, axis=1, keepdims=True) * (1.0 / 16.0)
        a1 = jnp.sum(q * ke, axis=1, keepdims=True) * (1.0 / 16.0)
        m = jnp.maximum(a0, a1)
        e0 = jnp.exp(a0 - m)
        e1 = jnp.exp(a1 - m)
        attn = (e0 * vp + e1 * ve) / (e0 + e1)

        rowid = r * 1024 + lax.broadcasted_iota(_i32, (1024, 1), 0)
        ae = jnp.where(rowid < S, attn, ve)

        final = jnp.tanh(
            jnp.dot(emb, wl1_ref[...], preferred_element_type=_f32)
            + jnp.dot(ae, wl2_ref[...], preferred_element_type=_f32)
            + bl_ref[...]
        )
        sym = (wlin_ref[...] + wlinT_ref[...]) * 0.5
        g_ref[...] = jnp.dot(final, sym, preferred_element_type=_f32)
        f_ref[...] = final

    whole = lambda shape: pl.BlockSpec(shape, lambda r: tuple(0 for _ in shape))
    return pl.pallas_call(
        body,
        grid=(R,),
        in_specs=[
            pl.BlockSpec((1024, 128), lambda r: (r, 0)),
            pl.BlockSpec((1024, 128), lambda r: (r, 0)),
            pl.BlockSpec((1024, 1), lambda r: (r, 0)),
            whole((1, H)),
            pl.BlockSpec((1024, H), lambda r: (r, 0)),
            whole((H, H)),
            whole((H, H)),
            whole((H, H)),
            whole((H, H)),
            whole((H, H)),
            whole((1, H)),
            whole((H, H)),
            whole((H, H)),
        ],
        out_specs=[
            pl.BlockSpec((1024, H), lambda r: (r, 0)),
            pl.BlockSpec((1024, H), lambda r: (r, 0)),
        ],
        out_shape=[
            jax.ShapeDtypeStruct((NP, H), _f32),
            jax.ShapeDtypeStruct((NP, H), _f32),
        ],
    )(a2lo, a2hi, dinv, b2r, prev_p, w_q, w_k, w_v, Wl1, Wl2, blr, wl, wlT)


def kernel(x, prev_embs, W_gc1, b_gc1, W_gc2, b_gc2, W_linear, b_linear,
           weight_lin, bias_lin, w_q, w_k, w_v, mp_adj, edges):
    src = mp_adj[0].astype(_i32)
    dst = mp_adj[1].astype(_i32)
    e0 = edges[:, 0].astype(_i32)
    e1 = edges[:, 1].astype(_i32)

    pad_e = EP - E
    src_p = jnp.concatenate([src, jnp.zeros((pad_e,), _i32)])
    dst_p = jnp.concatenate([dst, jnp.full((pad_e,), N, _i32)])
    e0_p = jnp.concatenate([e0, jnp.zeros((pad_e,), _i32)])
    e1_p = jnp.concatenate([e1, jnp.zeros((pad_e,), _i32)])

    x_p = jnp.concatenate([x, jnp.zeros((NP - N, F_IN), _f32)])
    prev_p = jnp.concatenate([prev_embs[0], jnp.zeros((NP - S, H), _f32)])

    deg = _sc_deg(dst_p)
    deg_col = deg.reshape(NP, 1)
    src2d = src_p.reshape(EP // 128, 128)
    dst2d = dst_p.reshape(EP // 128, 128)

    h1lo, h1hi, dinv = _tc1(x_p, W_gc1, deg_col)
    a1lo, a1hi = _sc_agg(h1lo, h1hi, src2d, dst2d)
    h2lo, h2hi = _tc2(a1lo, a1hi, dinv, b_gc1.reshape(1, H), W_gc2)
    a2lo, a2hi = _sc_agg(h2lo, h2hi, src2d, dst2d)
    g_tab, f_tab = _tc3(
        a2lo, a2hi, dinv, b_gc2.reshape(1, H), prev_p, w_q, w_k, w_v,
        W_linear[:H], W_linear[H:], b_linear.reshape(1, H),
        weight_lin, weight_lin.T,
    )
    out_p = _sc_edge(g_tab, f_tab, e0_p.reshape(EP // 80, 80),
                     e1_p.reshape(EP // 80, 80), bias_lin)
    return out_p[:E]
